# Initial kernel scaffold; baseline (speedup 1.0000x reference)
import functools

import jax
import jax.numpy as jnp
from jax import lax
from jax.experimental import pallas as pl
from jax.experimental.pallas import tpu as pltpu
from jax.experimental.pallas import tpu_sc as plsc

NN = 10000
NPAD = 10240
NC = 2
NS = 16
NW = NC * NS
ZN = NPAD // NS
F16 = 16

CHUNK = 2000
UNROLL = 4

_f32 = jnp.float32
_i32 = jnp.int32


def _mesh():
    return plsc.VectorSubcoreMesh(core_axis_name="c", subcore_axis_name="s")


def _make_deg_kernel(E):
    epw = E // NW
    nchunk = epw // CHUNK

    @functools.partial(
        pl.kernel,
        out_type=jax.ShapeDtypeStruct((NC, NPAD), _f32),
        mesh=_mesh(),
        scratch_types=[
            pltpu.VMEM((CHUNK,), _i32),
            pltpu.VMEM((CHUNK,), _f32),
            pltpu.VMEM_SHARED((NPAD,), _f32),
            pltpu.VMEM((ZN,), _f32),
        ],
    )
    def deg_kernel(col_hbm, ew_hbm, out_hbm, colv, ewv, deg_sh, zb):
        c = lax.axis_index("c")
        s = lax.axis_index("s")
        zero = jnp.zeros((16,), _f32)
        for j in range(ZN // 16):
            zb[pl.ds(j * 16, 16)] = zero
        pltpu.sync_copy(zb, deg_sh.at[pl.ds(s * ZN, ZN)])
        plsc.subcore_barrier()
        base = (c * NS + s) * epw
        for k in range(nchunk):
            off = base + k * CHUNK
            pltpu.sync_copy(col_hbm.at[pl.ds(off, CHUNK)], colv)
            pltpu.sync_copy(ew_hbm.at[pl.ds(off, CHUNK)], ewv)
            pltpu.sync_copy(ewv, deg_sh.at[colv], add=True)
        plsc.subcore_barrier()
        pltpu.sync_copy(deg_sh.at[pl.ds(s * ZN, ZN)],
                        out_hbm.at[c, pl.ds(s * ZN, ZN)])

    return deg_kernel


def _make_agg_kernel(E):
    epw = E // NW
    nchunk = epw // CHUNK

    @functools.partial(
        pl.kernel,
        out_type=jax.ShapeDtypeStruct((NC, NPAD, F16), _f32),
        mesh=_mesh(),
        scratch_types=[
            pltpu.VMEM((CHUNK,), _i32),
            pltpu.VMEM((CHUNK,), _i32),
            pltpu.VMEM((CHUNK,), _f32),
            pltpu.VMEM((CHUNK, F16), _f32),
            pltpu.VMEM_SHARED((NPAD, F16), _f32),
            pltpu.VMEM((ZN, F16), _f32),
            pltpu.SemaphoreType.DMA,
        ],
    )
    def agg_kernel(hs_hbm, row_hbm, col_hbm, ew_hbm, out_hbm,
                   rowv, colv, ewv, msg, acc_sh, zb, sem):
        c = lax.axis_index("c")
        s = lax.axis_index("s")
        zero = jnp.zeros((16,), _f32)

        def zbody(i, carry):
            zb[i, :] = zero
            return carry

        lax.fori_loop(0, ZN, zbody, 0)
        pltpu.sync_copy(zb, acc_sh.at[pl.ds(s * ZN, ZN)])
        plsc.subcore_barrier()

        base = (c * NS + s) * epw
        for k in range(nchunk):
            off = base + k * CHUNK
            pltpu.sync_copy(row_hbm.at[pl.ds(off, CHUNK)], rowv)
            pltpu.sync_copy(col_hbm.at[pl.ds(off, CHUNK)], colv)
            pltpu.sync_copy(ew_hbm.at[pl.ds(off, CHUNK)], ewv)
            pltpu.async_copy(hs_hbm.at[rowv], msg, sem).wait()

            def sbody(i, carry):
                for u in range(UNROLL):
                    j = i * UNROLL + u
                    msg[j, :] = msg[j, :] * ewv[j]
                return carry

            lax.fori_loop(0, CHUNK // UNROLL, sbody, 0)
            pltpu.sync_copy(msg, acc_sh.at[colv], add=True)
        plsc.subcore_barrier()
        pltpu.sync_copy(acc_sh.at[pl.ds(s * ZN, ZN)],
                        out_hbm.at[c, pl.ds(s * ZN, ZN)])

    return agg_kernel


def _t_dinv(degp_ref, o_ref):
    o_ref[:] = lax.rsqrt(degp_ref[0] + degp_ref[1] + 1.0)


def _t_lin1(x_ref, w_ref, d_ref, o_ref):
    h = jnp.dot(x_ref[:], w_ref[:], preferred_element_type=_f32)
    o_ref[:] = d_ref[:] * h


def _t_mid(sp_ref, hs_ref, d_ref, b_ref, w_ref, o_ref):
    z = d_ref[:] * (sp_ref[0, :NN, :] + sp_ref[1, :NN, :] + hs_ref[:]) + b_ref[:]
    z = jnp.maximum(z, 0.0)
    o_ref[:] = d_ref[:] * jnp.dot(z, w_ref[:], preferred_element_type=_f32)


def _t_final(sp_ref, hs_ref, d_ref, b_ref, o_ref):
    z = d_ref[:] * (sp_ref[0, :NN, :] + sp_ref[1, :NN, :] + hs_ref[:]) + b_ref[:]
    m = jnp.max(z, axis=1, keepdims=True)
    e = jnp.exp(z - m)
    lse = jnp.log(jnp.sum(e, axis=1, keepdims=True)) + m
    o_ref[:] = z - lse


def kernel(x, edge_index, edge_weight, W1, b1, W2, b2):
    E = edge_index.shape[1]
    row = edge_index[0]
    col = edge_index[1]

    deg_kernel = _make_deg_kernel(E)
    agg_kernel = _make_agg_kernel(E)

    degp = deg_kernel(col, edge_weight)
    dinv = pl.pallas_call(
        _t_dinv,
        out_shape=jax.ShapeDtypeStruct((NPAD // 128, 128), _f32),
    )(degp.reshape(NC, NPAD // 128, 128))
    dinv_col = dinv.reshape(NPAD, 1)[:NN]

    hs1 = pl.pallas_call(
        _t_lin1,
        out_shape=jax.ShapeDtypeStruct((NN, F16), _f32),
    )(x, W1, dinv_col)

    s1p = agg_kernel(hs1, row, col, edge_weight)

    hs2 = pl.pallas_call(
        _t_mid,
        out_shape=jax.ShapeDtypeStruct((NN, F16), _f32),
    )(s1p, hs1, dinv_col, b1.reshape(1, F16), W2)

    s2p = agg_kernel(hs2, row, col, edge_weight)

    out = pl.pallas_call(
        _t_final,
        out_shape=jax.ShapeDtypeStruct((NN, F16), _f32),
    )(s2p, hs2, dinv_col, b2.reshape(1, F16))
    return out


# speedup vs baseline: 48.6575x; 48.6575x over previous
import functools

import jax
import jax.numpy as jnp
from jax import lax
from jax.experimental import pallas as pl
from jax.experimental.pallas import tpu as pltpu
from jax.experimental.pallas import tpu_sc as plsc

NN = 10000
NPAD = 10240
NC = 2
NS = 16
NW = NC * NS
ZN = NPAD // NS
F16 = 16

CHUNK = 2000
UNROLL = 4

_f32 = jnp.float32
_i32 = jnp.int32


def _mesh():
    return plsc.VectorSubcoreMesh(core_axis_name="c", subcore_axis_name="s")


def _make_deg_kernel(E):
    epw = E // NW
    nchunk = epw // CHUNK

    @functools.partial(
        pl.kernel,
        out_type=jax.ShapeDtypeStruct((NC, NPAD), _f32),
        mesh=_mesh(),
        scratch_types=[
            pltpu.VMEM((CHUNK,), _i32),
            pltpu.VMEM((CHUNK,), _f32),
            pltpu.VMEM_SHARED((NPAD,), _f32),
            pltpu.VMEM((ZN,), _f32),
        ],
    )
    def deg_kernel(col_hbm, ew_hbm, out_hbm, colv, ewv, deg_sh, zb):
        c = lax.axis_index("c")
        s = lax.axis_index("s")
        zero = jnp.zeros((16,), _f32)
        for j in range(ZN // 16):
            zb[pl.ds(j * 16, 16)] = zero
        pltpu.sync_copy(zb, deg_sh.at[pl.ds(s * ZN, ZN)])
        plsc.subcore_barrier()
        base = (c * NS + s) * epw
        for k in range(nchunk):
            off = base + k * CHUNK
            pltpu.sync_copy(col_hbm.at[pl.ds(off, CHUNK)], colv)
            pltpu.sync_copy(ew_hbm.at[pl.ds(off, CHUNK)], ewv)
            pltpu.sync_copy(ewv, deg_sh.at[colv], add=True)
        plsc.subcore_barrier()
        pltpu.sync_copy(deg_sh.at[pl.ds(s * ZN, ZN)],
                        out_hbm.at[c, pl.ds(s * ZN, ZN)])

    return deg_kernel


def _make_agg_kernel(E):
    epw = E // NW
    nchunk = epw // CHUNK

    @functools.partial(
        pl.kernel,
        out_type=jax.ShapeDtypeStruct((NC, NPAD, F16), _f32),
        mesh=_mesh(),
        scratch_types=[
            pltpu.VMEM((CHUNK,), _i32),
            pltpu.VMEM((CHUNK,), _i32),
            pltpu.VMEM((CHUNK,), _f32),
            pltpu.VMEM((CHUNK, F16), _f32),
            pltpu.VMEM_SHARED((NPAD, F16), _f32),
            pltpu.VMEM((ZN, F16), _f32),
            pltpu.SemaphoreType.DMA,
        ],
        compiler_params=pltpu.CompilerParams(use_tc_tiling_on_sc=False),
    )
    def agg_kernel(hs_hbm, row_hbm, col_hbm, ew_hbm, out_hbm,
                   rowv, colv, ewv, msg, acc_sh, zb, sem):
        c = lax.axis_index("c")
        s = lax.axis_index("s")
        zero = jnp.zeros((16,), _f32)

        def zbody(i, carry):
            zb[i, :] = zero
            return carry

        lax.fori_loop(0, ZN, zbody, 0)
        pltpu.sync_copy(zb, acc_sh.at[pl.ds(s * ZN, ZN)])
        plsc.subcore_barrier()

        base = (c * NS + s) * epw
        for k in range(nchunk):
            off = base + k * CHUNK
            pltpu.sync_copy(row_hbm.at[pl.ds(off, CHUNK)], rowv)
            pltpu.sync_copy(col_hbm.at[pl.ds(off, CHUNK)], colv)
            pltpu.sync_copy(ew_hbm.at[pl.ds(off, CHUNK)], ewv)
            pltpu.async_copy(hs_hbm.at[rowv], msg, sem).wait()

            def sbody(i, carry):
                w16 = ewv[pl.ds(i * 16, 16)]
                for u in range(16):
                    j = i * 16 + u
                    msg[j, :] = msg[j, :] * w16[u]
                return carry

            lax.fori_loop(0, CHUNK // 16, sbody, 0)
            pltpu.sync_copy(msg, acc_sh.at[colv], add=True)
        plsc.subcore_barrier()
        pltpu.sync_copy(acc_sh.at[pl.ds(s * ZN, ZN)],
                        out_hbm.at[c, pl.ds(s * ZN, ZN)])

    return agg_kernel


def _t_dinv(degp_ref, o_ref):
    o_ref[:] = lax.rsqrt(degp_ref[0] + degp_ref[1] + 1.0)


def _t_lin1(x_ref, w_ref, d_ref, o_ref):
    h = jnp.dot(x_ref[:], w_ref[:], preferred_element_type=_f32)
    o_ref[:] = d_ref[:] * h


def _t_mid(sp_ref, hs_ref, d_ref, b_ref, w_ref, o_ref):
    z = d_ref[:] * (sp_ref[0, :NN, :] + sp_ref[1, :NN, :] + hs_ref[:]) + b_ref[:]
    z = jnp.maximum(z, 0.0)
    o_ref[:] = d_ref[:] * jnp.dot(z, w_ref[:], preferred_element_type=_f32)


def _t_final(sp_ref, hs_ref, d_ref, b_ref, o_ref):
    z = d_ref[:] * (sp_ref[0, :NN, :] + sp_ref[1, :NN, :] + hs_ref[:]) + b_ref[:]
    m = jnp.max(z, axis=1, keepdims=True)
    e = jnp.exp(z - m)
    lse = jnp.log(jnp.sum(e, axis=1, keepdims=True)) + m
    o_ref[:] = z - lse


def kernel(x, edge_index, edge_weight, W1, b1, W2, b2):
    E = edge_index.shape[1]
    row = edge_index[0]
    col = edge_index[1]

    deg_kernel = _make_deg_kernel(E)
    agg_kernel = _make_agg_kernel(E)

    degp = deg_kernel(col, edge_weight)
    dinv = pl.pallas_call(
        _t_dinv,
        out_shape=jax.ShapeDtypeStruct((NPAD // 128, 128), _f32),
    )(degp.reshape(NC, NPAD // 128, 128))
    dinv_col = dinv.reshape(NPAD, 1)[:NN]

    hs1 = pl.pallas_call(
        _t_lin1,
        out_shape=jax.ShapeDtypeStruct((NN, F16), _f32),
    )(x, W1, dinv_col)

    s1p = agg_kernel(hs1, row, col, edge_weight)

    hs2 = pl.pallas_call(
        _t_mid,
        out_shape=jax.ShapeDtypeStruct((NN, F16), _f32),
    )(s1p, hs1, dinv_col, b1.reshape(1, F16), W2)

    s2p = agg_kernel(hs2, row, col, edge_weight)

    out = pl.pallas_call(
        _t_final,
        out_shape=jax.ShapeDtypeStruct((NN, F16), _f32),
    )(s2p, hs2, dinv_col, b2.reshape(1, F16))
    return out


# speedup vs baseline: 65.8026x; 1.3524x over previous
import functools

import jax
import jax.numpy as jnp
from jax import lax
from jax.experimental import pallas as pl
from jax.experimental.pallas import tpu as pltpu
from jax.experimental.pallas import tpu_sc as plsc

NN = 10000
NPAD = 10240
NC = 2
NS = 16
NW = NC * NS
ZN = NPAD // NS
F16 = 16

CHUNK = 2000
NBUF = 3

_f32 = jnp.float32
_i32 = jnp.int32


def _mesh():
    return plsc.VectorSubcoreMesh(core_axis_name="c", subcore_axis_name="s")


def _make_deg_kernel(E):
    epw = E // NW
    nchunk = epw // CHUNK

    @functools.partial(
        pl.kernel,
        out_type=jax.ShapeDtypeStruct((NC, NPAD), _f32),
        mesh=_mesh(),
        scratch_types=[
            pltpu.VMEM((CHUNK,), _i32),
            pltpu.VMEM((CHUNK,), _f32),
            pltpu.VMEM_SHARED((NPAD,), _f32),
            pltpu.VMEM((ZN,), _f32),
        ],
    )
    def deg_kernel(ei_hbm, ew_hbm, out_hbm, colv, ewv, deg_sh, zb):
        c = lax.axis_index("c")
        s = lax.axis_index("s")
        zero = jnp.zeros((16,), _f32)
        for j in range(ZN // 16):
            zb[pl.ds(j * 16, 16)] = zero
        pltpu.sync_copy(zb, deg_sh.at[pl.ds(s * ZN, ZN)])
        plsc.subcore_barrier()
        base = (c * NS + s) * epw
        for k in range(nchunk):
            off = base + k * CHUNK
            pltpu.sync_copy(ei_hbm.at[pl.ds(E + off, CHUNK)], colv)
            pltpu.sync_copy(ew_hbm.at[pl.ds(off, CHUNK)], ewv)
            pltpu.sync_copy(ewv, deg_sh.at[colv], add=True)
        plsc.subcore_barrier()
        pltpu.sync_copy(deg_sh.at[pl.ds(s * ZN, ZN)],
                        out_hbm.at[c, pl.ds(s * ZN, ZN)])

    return deg_kernel


def _make_agg_kernel(E):
    epw = E // NW
    nchunk = epw // CHUNK

    scratch = (
        [pltpu.VMEM((CHUNK,), _i32)] * NBUF +
        [pltpu.VMEM((CHUNK,), _i32)] * NBUF +
        [pltpu.VMEM((CHUNK,), _f32)] * NBUF +
        [pltpu.VMEM((CHUNK, F16), _f32)] * NBUF +
        [pltpu.VMEM_SHARED((NPAD, F16), _f32),
         pltpu.VMEM((64, F16), _f32)] +
        [pltpu.SemaphoreType.DMA] * (3 * NBUF)
    )

    @functools.partial(
        pl.kernel,
        out_type=jax.ShapeDtypeStruct((NC, NPAD, F16), _f32),
        mesh=_mesh(),
        scratch_types=scratch,
        compiler_params=pltpu.CompilerParams(use_tc_tiling_on_sc=False),
    )
    def agg_kernel(hs_hbm, ei_hbm, ew_hbm, out_hbm, *bufs):
        rowv = bufs[0:NBUF]
        colv = bufs[NBUF:2 * NBUF]
        ewv = bufs[2 * NBUF:3 * NBUF]
        msg = bufs[3 * NBUF:4 * NBUF]
        acc_sh, zb = bufs[4 * NBUF], bufs[4 * NBUF + 1]
        sem_cp = bufs[4 * NBUF + 2:4 * NBUF + 2 + NBUF]
        sem_g = bufs[4 * NBUF + 2 + NBUF:4 * NBUF + 2 + 2 * NBUF]
        sem_s = bufs[4 * NBUF + 2 + 2 * NBUF:4 * NBUF + 2 + 3 * NBUF]

        c = lax.axis_index("c")
        s = lax.axis_index("s")
        zero = jnp.zeros((16,), _f32)

        def zbody(i, carry):
            zb[i, :] = zero
            return carry

        lax.fori_loop(0, 64, zbody, 0)
        for j in range(ZN // 64):
            pltpu.sync_copy(zb, acc_sh.at[pl.ds(s * ZN + j * 64, 64)])
        plsc.subcore_barrier()

        base = (c * NS + s) * epw

        def issue_copies(k):
            b = k % NBUF
            off = base + k * CHUNK
            r = pltpu.async_copy(ei_hbm.at[pl.ds(off, CHUNK)], rowv[b],
                                 sem_cp[b])
            cc = pltpu.async_copy(ei_hbm.at[pl.ds(E + off, CHUNK)], colv[b],
                                  sem_cp[b])
            w = pltpu.async_copy(ew_hbm.at[pl.ds(off, CHUNK)], ewv[b],
                                 sem_cp[b])
            return r, cc, w

        def issue_gather(k, cps):
            for d in cps:
                d.wait()
            b = k % NBUF
            return pltpu.async_copy(hs_hbm.at[rowv[b]], msg[b], sem_g[b])

        def scale(k):
            b = k % NBUF
            mb = msg[b]
            wb = ewv[b]

            def sbody(i, carry):
                w16 = wb[pl.ds(i * 16, 16)]
                for u in range(16):
                    j = i * 16 + u
                    mb[j, :] = mb[j, :] * w16[u]
                return carry

            lax.fori_loop(0, CHUNK // 16, sbody, 0)

        def issue_scatter(k):
            b = k % NBUF
            return pltpu.async_copy(msg[b], acc_sh.at[colv[b]], sem_s[b],
                                    add=True)

        cps = [None] * nchunk
        gth = [None] * nchunk
        sct = [None] * nchunk
        cps[0] = issue_copies(0)
        if nchunk > 1:
            cps[1] = issue_copies(1)
        gth[0] = issue_gather(0, cps[0])
        for k in range(nchunk):
            gth[k].wait()
            if k + 1 < nchunk:
                gth[k + 1] = issue_gather(k + 1, cps[k + 1])
            if k + 2 < nchunk:
                if k + 2 >= NBUF:
                    sct[k + 2 - NBUF].wait()
                cps[k + 2] = issue_copies(k + 2)
            scale(k)
            sct[k] = issue_scatter(k)
        for k in range(max(0, nchunk - NBUF), nchunk):
            if sct[k] is not None:
                sct[k].wait()
        plsc.subcore_barrier()
        pltpu.sync_copy(acc_sh.at[pl.ds(s * ZN, ZN)],
                        out_hbm.at[c, pl.ds(s * ZN, ZN)])

    return agg_kernel


def _t_lin1(x_ref, w_ref, degp_ref, hs_ref, dinv_ref):
    dinv = lax.rsqrt(degp_ref[0] + degp_ref[1] + 1.0)
    dinv_ref[:] = dinv
    h = jnp.dot(x_ref[:], w_ref[:], preferred_element_type=_f32)
    hs_ref[:] = dinv[:NN] * h


def _t_mid(sp_ref, hs_ref, d_ref, b_ref, w_ref, o_ref):
    d = d_ref[:NN]
    z = d * (sp_ref[0, :NN, :] + sp_ref[1, :NN, :] + hs_ref[:]) + b_ref[:]
    z = jnp.maximum(z, 0.0)
    o_ref[:] = d * jnp.dot(z, w_ref[:], preferred_element_type=_f32)


def _t_final(sp_ref, hs_ref, d_ref, b_ref, o_ref):
    z = d_ref[:NN] * (sp_ref[0, :NN, :] + sp_ref[1, :NN, :] + hs_ref[:]) + b_ref[:]
    m = jnp.max(z, axis=1, keepdims=True)
    e = jnp.exp(z - m)
    lse = jnp.log(jnp.sum(e, axis=1, keepdims=True)) + m
    o_ref[:] = z - lse


def kernel(x, edge_index, edge_weight, W1, b1, W2, b2):
    E = edge_index.shape[1]

    deg_kernel = _make_deg_kernel(E)
    agg_kernel = _make_agg_kernel(E)

    ei_flat = edge_index.reshape(2 * E)
    degp = deg_kernel(ei_flat, edge_weight)

    hs1, dinv = pl.pallas_call(
        _t_lin1,
        out_shape=[jax.ShapeDtypeStruct((NN, F16), _f32),
                   jax.ShapeDtypeStruct((NPAD, 1), _f32)],
    )(x, W1, degp.reshape(NC, NPAD, 1))

    s1p = agg_kernel(hs1, ei_flat, edge_weight)

    hs2 = pl.pallas_call(
        _t_mid,
        out_shape=jax.ShapeDtypeStruct((NN, F16), _f32),
    )(s1p, hs1, dinv, b1.reshape(1, F16), W2)

    s2p = agg_kernel(hs2, ei_flat, edge_weight)

    out = pl.pallas_call(
        _t_final,
        out_shape=jax.ShapeDtypeStruct((NN, F16), _f32),
    )(s2p, hs2, dinv, b2.reshape(1, F16))
    return out
